# baseline (speedup 1.0000x reference)
import functools

import jax
import jax.numpy as jnp
from jax import lax
from jax.experimental import pallas as pl
from jax.experimental.pallas import tpu as pltpu
from jax.experimental.pallas import tpu_sc as plsc

N = 10000
E = 160000
DIN = 256
DH = 512
DOUT = 256
MAXDEG = 128

NH = 10240
ADJ_SIZE = N * MAXDEG + 128
E1P = 163840
RPT1 = 80
E2P = 327680
RPT2 = 160

_INTERPRET = False


def _zero_ref(ref, nwords):
    z = jnp.zeros((16,), ref.dtype)

    def body(i, _):
        ref[pl.ds(i * 16, 16)] = z
        return 0

    lax.fori_loop(0, nwords // 16, body, 0)


def _build_graph(src2_h, dst2_h, adj_h, deg_h, rpt, vr, sid,
                 src2, dst2, rank2, fidx2, hist, tmp, degacc, sbuf,
                 shared, shared2, sem):
    nv = rpt * 8
    iota = lax.iota(jnp.int32, 16)

    cp_s = pltpu.async_copy(src2_h.at[pl.ds(sid * rpt, rpt)],
                            src2.at[pl.ds(0, rpt)], sem)
    cp_d = pltpu.async_copy(dst2_h.at[pl.ds(sid * rpt, rpt)],
                            dst2.at[pl.ds(0, rpt)], sem)
    cp_s.wait()
    cp_d.wait()

    _zero_ref(hist, NH)

    def p1_one(i):
        row = i // 8
        col = (i % 8) * 16
        s = src2[row, pl.ds(col, 16)]
        ks, lane = plsc.sort_key_val(s * 16 + iota, iota)
        ss = ks >> 4
        sbuf[...] = ss
        prev = plsc.load_gather(sbuf, [jnp.maximum(iota - 1, 0)])
        nxt = plsc.load_gather(sbuf, [jnp.minimum(iota + 1, 15)])
        is_start = (iota == 0) | (ss != prev)
        is_end = (iota == 15) | (ss != nxt)
        sidx = plsc.cummax(jnp.where(is_start, iota, 0))
        runpos = iota - sidx
        old = plsc.load_gather(hist, [ss])
        ranks = old + runpos
        plsc.store_scatter(hist, [ss], ranks + 1, mask=is_end)
        rowv = jnp.zeros((16,), jnp.int32) + row
        plsc.store_scatter(rank2, [rowv, col + lane], ranks)

    def p1(i, _):
        p1_one(2 * i)
        p1_one(2 * i + 1)
        return 0

    lax.fori_loop(0, nv // 4, p1, 0)

    pltpu.sync_copy(hist, shared.at[sid])
    plsc.subcore_barrier()
    _zero_ref(degacc, 640)
    for t in range(16):
        pltpu.sync_copy(degacc, shared2.at[t, pl.ds(sid * 640, 640)])
        pltpu.sync_copy(shared.at[t, pl.ds(sid * 640, 640)],
                        tmp.at[pl.ds(0, 640)])
        for j in range(640 // 16):
            degacc[pl.ds(j * 16, 16)] = (degacc[pl.ds(j * 16, 16)]
                                         + tmp[pl.ds(j * 16, 16)])
    for j in range(640 // 16):
        degacc[pl.ds(j * 16, 16)] = jnp.minimum(degacc[pl.ds(j * 16, 16)],
                                                MAXDEG)
    pltpu.sync_copy(degacc, deg_h.at[pl.ds(sid * 640, 640)])
    plsc.subcore_barrier()
    pltpu.sync_copy(shared2.at[sid], hist)

    def p3(i, _):
        row = i // 8
        col = (i % 8) * 16
        s = src2[row, pl.ds(col, 16)]
        rl = rank2[row, pl.ds(col, 16)]
        rank = rl + plsc.load_gather(hist, [s])
        valid = (s < N) & (rank < MAXDEG)
        f = jnp.where(valid, s * MAXDEG + rank, N * MAXDEG + iota)
        fidx2[row, pl.ds(col, 16)] = f
        return 0

    lax.fori_loop(0, nv, p3, 0)

    nrow = jnp.clip(vr - sid * rpt, 0, rpt)
    ngrp = nrow // 8

    def pgrp(g, _):
        cps = [pltpu.async_copy(dst2.at[g * 8 + b],
                                adj_h.at[fidx2.at[g * 8 + b]], sem)
               for b in range(8)]
        for cp in cps:
            cp.wait()
        return 0

    lax.fori_loop(0, ngrp, pgrp, 0)

    def prem(j, _):
        pltpu.async_copy(dst2.at[ngrp * 8 + j],
                         adj_h.at[fidx2.at[ngrp * 8 + j]], sem).wait()
        return 0

    lax.fori_loop(0, nrow - ngrp * 8, prem, 0)


def _a1_body(srcb2_h, dstb2_h, srcd2_h, dstd2_h,
             adjb_h, degb_h, adjd_h, degd_h,
             src2, dst2, rank2, fidx2, hist, tmp, degacc, sbuf,
             shared, shared2, sem):
    cid = lax.axis_index("c")
    sid = lax.axis_index("s")

    @pl.when(cid == 0)
    def _():
        _build_graph(srcb2_h, dstb2_h, adjb_h, degb_h, RPT2, 2 * E // 128, sid,
                     src2, dst2, rank2, fidx2, hist, tmp, degacc, sbuf,
                     shared, shared2, sem)

    @pl.when(cid == 1)
    def _():
        _build_graph(srcd2_h, dstd2_h, adjd_h, degd_h, RPT1, E // 128, sid,
                     src2, dst2, rank2, fidx2, hist, tmp, degacc, sbuf,
                     shared, shared2, sem)


_a1_call = functools.partial(
    pl.kernel,
    out_type=(
        jax.ShapeDtypeStruct((ADJ_SIZE,), jnp.int32),
        jax.ShapeDtypeStruct((NH,), jnp.int32),
        jax.ShapeDtypeStruct((ADJ_SIZE,), jnp.int32),
        jax.ShapeDtypeStruct((NH,), jnp.int32),
    ),
    mesh=plsc.VectorSubcoreMesh(core_axis_name="c", subcore_axis_name="s"),
    scratch_types=[
        pltpu.VMEM((RPT2, 128), jnp.int32),
        pltpu.VMEM((RPT2, 128), jnp.int32),
        pltpu.VMEM((RPT2, 128), jnp.int32),
        pltpu.VMEM((RPT2, 128), jnp.int32),
        pltpu.VMEM((NH,), jnp.int32),
        pltpu.VMEM((NH,), jnp.int32),
        pltpu.VMEM((640,), jnp.int32),
        pltpu.VMEM((16,), jnp.int32),
        pltpu.VMEM_SHARED((16, NH), jnp.int32),
        pltpu.VMEM_SHARED((16, NH), jnp.int32),
        pltpu.SemaphoreType.DMA,
    ],
    compiler_params=pltpu.CompilerParams(needs_layout_passes=False),
)(_a1_body)


def _a2_body(adjb_h, degb_h, adjd_h, degd_h, start2_h, r2_h, r1_h, x_h,
             xg_h, wl_h, coef_h,
             degtab, n2buf, coefbuf, startbuf, rbuf, nxtbuf, ubuf, wbuf,
             ustage, idx48, rows, fbuf, shared_n2, shared_u, sem):
    cid = lax.axis_index("c")
    sid = lax.axis_index("s")
    iota = lax.iota(jnp.int32, 16)

    @pl.when((cid == 0) & (sid == 0))
    def _():
        pltpu.sync_copy(degd_h, degtab)
        pltpu.sync_copy(start2_h, startbuf)
        _zero_ref(coefbuf, 64)
        cur = startbuf[...]
        mask = iota < 10
        one = jnp.full((16,), 1.0, jnp.float32)
        idx0 = jnp.where(mask, iota, 40 + iota - 10)
        plsc.store_scatter(n2buf, [idx0], cur)
        plsc.store_scatter(coefbuf, [idx0], one, mask=mask)
        for t in range(1, 4):
            pltpu.sync_copy(r2_h.at[pl.ds((t - 1) * 16, 16)], rbuf)
            r = rbuf[...]
            d = plsc.load_gather(degtab, [cur])
            has = d > 0
            idx = lax.rem(r, jnp.maximum(d, 1))
            fbuf[...] = cur * MAXDEG + idx
            pltpu.async_copy(adjd_h.at[fbuf], nxtbuf, sem).wait()
            mask = mask & has
            cur = jnp.where(mask, nxtbuf[...], cur)
            idx_t = jnp.where(iota < 10, t * 10 + iota, 40 + iota - 10)
            plsc.store_scatter(n2buf, [idx_t], cur)
            plsc.store_scatter(coefbuf, [idx_t],
                               jnp.where(mask, 1.0, 0.0), mask=iota < 10)
        pltpu.sync_copy(n2buf, shared_n2)
        pltpu.sync_copy(coefbuf.at[pl.ds(0, 48)], coef_h)

    plsc.subcore_barrier()

    @pl.when((cid == 0) & (sid < 13))
    def _():
        pltpu.sync_copy(shared_n2, n2buf)
        pltpu.sync_copy(degb_h, degtab)
        sp = sid * 16 + iota
        valid = sp < 200
        slot = jnp.minimum(sp // 5, 39)
        path = sp % 5
        v = plsc.load_gather(n2buf, [slot])
        ubuf[...] = v
        pltpu.sync_copy(ubuf, shared_u.at[pl.ds(sid * 16, 16)])
        cur = v
        mask = valid
        for t in range(3):
            fbuf[...] = t * 50000 + path * N + v
            pltpu.async_copy(r1_h.at[fbuf], rbuf, sem).wait()
            r = rbuf[...]
            d = plsc.load_gather(degtab, [cur])
            has = d > 0
            idx = lax.rem(r, jnp.maximum(d, 1))
            fbuf[...] = cur * MAXDEG + idx
            pltpu.async_copy(adjb_h.at[fbuf], nxtbuf, sem).wait()
            mask = mask & has
            cur = jnp.where(mask, nxtbuf[...], cur)
            ubuf[...] = cur
            pltpu.sync_copy(ubuf, shared_u.at[pl.ds((1 + t) * 256 + sid * 16, 16)])
            wbuf[...] = jnp.where(mask, 1.0, 0.0)
            pltpu.sync_copy(wbuf, wl_h.at[pl.ds(t * 256 + sid * 16, 16)])

    plsc.subcore_barrier()

    @pl.when(cid == 0)
    def _():
        pltpu.sync_copy(shared_u, ustage)
        for b in range(3):
            jv = sid * 48 + b * 16 + iota
            slot = jnp.minimum(jv >> 4, 39)
            k = jv & 15
            km1 = jnp.maximum(k - 1, 0)
            pos = jnp.where(k == 0, slot * 5,
                            (1 + lax.rem(km1, 3)) * 256 + slot * 5 + km1 // 3)
            vals = plsc.load_gather(ustage, [pos])
            idx48[pl.ds(b * 16, 16)] = vals
        pltpu.async_copy(x_h.at[idx48], rows, sem).wait()
        pltpu.sync_copy(rows, xg_h.at[pl.ds(sid * 48, 48)])


_a2_call = functools.partial(
    pl.kernel,
    out_type=(
        jax.ShapeDtypeStruct((768, DIN), jnp.float32),
        jax.ShapeDtypeStruct((768,), jnp.float32),
        jax.ShapeDtypeStruct((48,), jnp.float32),
    ),
    mesh=plsc.VectorSubcoreMesh(core_axis_name="c", subcore_axis_name="s"),
    scratch_types=[
        pltpu.VMEM((NH,), jnp.int32),
        pltpu.VMEM((64,), jnp.int32),
        pltpu.VMEM((64,), jnp.float32),
        pltpu.VMEM((16,), jnp.int32),
        pltpu.VMEM((16,), jnp.int32),
        pltpu.VMEM((16,), jnp.int32),
        pltpu.VMEM((16,), jnp.int32),
        pltpu.VMEM((16,), jnp.float32),
        pltpu.VMEM((1024,), jnp.int32),
        pltpu.VMEM((48,), jnp.int32),
        pltpu.VMEM((48, DIN), jnp.float32),
        pltpu.VMEM((16,), jnp.int32),
        pltpu.VMEM_SHARED((64,), jnp.int32),
        pltpu.VMEM_SHARED((1024,), jnp.int32),
        pltpu.SemaphoreType.DMA,
    ],
    compiler_params=pltpu.CompilerParams(needs_layout_passes=False),
)(_a2_body)


def _a2w2_body(adjd_h, degd_h, start2_h, r2_h, n2_h, coef_h,
               degtab, n2buf, coefbuf, startbuf, rbuf, nxtbuf, fbuf, sem):
    cid = lax.axis_index("c")
    sid = lax.axis_index("s")
    iota = lax.iota(jnp.int32, 16)

    @pl.when((cid == 0) & (sid == 0))
    def _():
        pltpu.sync_copy(degd_h, degtab)
        pltpu.sync_copy(start2_h, startbuf)
        _zero_ref(coefbuf, 64)
        cur = startbuf[...]
        mask = iota < 10
        one = jnp.full((16,), 1.0, jnp.float32)
        idx0 = jnp.where(mask, iota, 40 + iota - 10)
        plsc.store_scatter(n2buf, [idx0], cur)
        plsc.store_scatter(coefbuf, [idx0], one, mask=mask)
        for t in range(1, 4):
            pltpu.sync_copy(r2_h.at[pl.ds((t - 1) * 16, 16)], rbuf)
            r = rbuf[...]
            d = plsc.load_gather(degtab, [cur])
            has = d > 0
            idx = lax.rem(r, jnp.maximum(d, 1))
            fbuf[...] = cur * MAXDEG + idx
            pltpu.async_copy(adjd_h.at[fbuf], nxtbuf, sem).wait()
            mask = mask & has
            cur = jnp.where(mask, nxtbuf[...], cur)
            idx_t = jnp.where(iota < 10, t * 10 + iota, 40 + iota - 10)
            plsc.store_scatter(n2buf, [idx_t], cur)
            plsc.store_scatter(coefbuf, [idx_t],
                               jnp.where(mask, 1.0, 0.0), mask=iota < 10)
        pltpu.sync_copy(n2buf, n2_h)
        pltpu.sync_copy(coefbuf.at[pl.ds(0, 48)], coef_h)


_a2w2_call = functools.partial(
    pl.kernel,
    out_type=(
        jax.ShapeDtypeStruct((64,), jnp.int32),
        jax.ShapeDtypeStruct((48,), jnp.float32),
    ),
    mesh=plsc.VectorSubcoreMesh(core_axis_name="c", subcore_axis_name="s"),
    scratch_types=[
        pltpu.VMEM((NH,), jnp.int32),
        pltpu.VMEM((64,), jnp.int32),
        pltpu.VMEM((64,), jnp.float32),
        pltpu.VMEM((16,), jnp.int32),
        pltpu.VMEM((16,), jnp.int32),
        pltpu.VMEM((16,), jnp.int32),
        pltpu.VMEM((16,), jnp.int32),
        pltpu.SemaphoreType.DMA,
    ],
    compiler_params=pltpu.CompilerParams(needs_layout_passes=False),
)(_a2w2_body)


def _a2t_body(u_h, x_h, xg_h, idx48, rows, sem):
    cid = lax.axis_index("c")
    sid = lax.axis_index("s")

    @pl.when(cid == 0)
    def _():
        pltpu.sync_copy(u_h.at[pl.ds(sid * 48, 48)], idx48)
        pltpu.async_copy(x_h.at[idx48], rows, sem).wait()
        pltpu.sync_copy(rows, xg_h.at[pl.ds(sid * 48, 48)])


_a2t_call = functools.partial(
    pl.kernel,
    out_type=(jax.ShapeDtypeStruct((768, DIN), jnp.float32),),
    mesh=plsc.VectorSubcoreMesh(core_axis_name="c", subcore_axis_name="s"),
    scratch_types=[
        pltpu.VMEM((48,), jnp.int32),
        pltpu.VMEM((48, DIN), jnp.float32),
        pltpu.SemaphoreType.DMA,
    ],
    compiler_params=pltpu.CompilerParams(needs_layout_passes=False),
)(_a2t_body)


def _tail_body(xg_ref, w1_ref, b1_ref, w3_ref, b3_ref, wt_ref, coefn_ref,
               out_ref):
    xg = xg_ref[...][0:640]
    h = jnp.maximum(
        jnp.dot(xg, w1_ref[...], preferred_element_type=jnp.float32)
        + b1_ref[...], 0.0)
    wt = wt_ref[...]
    cnt = jnp.sum(wt, axis=1, keepdims=True)
    wn = (wt / cnt)[:, :, None]
    a = jnp.sum(h.reshape(40, 16, DH) * wn, axis=1)
    o = jnp.maximum(
        jnp.dot(a, w3_ref[...], preferred_element_type=jnp.float32)
        + b3_ref[...], 0.0)
    coef = coefn_ref[...]
    den = jnp.sum(coef)
    out_ref[...] = jnp.sum(o * (coef / den), axis=0, keepdims=True)


def _tail(xg, w1, b1, w3, b3, wt, coefn):
    return pl.pallas_call(
        _tail_body,
        out_shape=jax.ShapeDtypeStruct((1, DOUT), jnp.float32),
        interpret=_INTERPRET,
    )(xg, w1, b1, w3, b3, wt, coefn)


def kernel(x, edge_index, edge_attr, W1, b1, W2, b2, W3, b3):
    src = edge_index[0].astype(jnp.int32)
    dst = edge_index[1].astype(jnp.int32)
    keep = src != dst
    src_bi = jnp.concatenate([src, jnp.where(keep, dst, jnp.int32(N))])
    dst_bi = jnp.concatenate([dst, jnp.where(keep, src, jnp.int32(0))])

    pad1 = jnp.full((E1P - E,), N, jnp.int32)
    zpad1 = jnp.zeros((E1P - E,), jnp.int32)
    pad2 = jnp.full((E2P - 2 * E,), N, jnp.int32)
    zpad2 = jnp.zeros((E2P - 2 * E,), jnp.int32)
    srcd2 = jnp.concatenate([src, pad1]).reshape(E1P // 128, 128)
    dstd2 = jnp.concatenate([dst, zpad1]).reshape(E1P // 128, 128)
    srcb2 = jnp.concatenate([src_bi, pad2]).reshape(E2P // 128, 128)
    dstb2 = jnp.concatenate([dst_bi, zpad2]).reshape(E2P // 128, 128)

    adjb, degb, adjd, degd = _a1_call(srcb2, dstb2, srcd2, dstd2)

    k = jax.random.key(1)
    r1 = []
    for _ in range(3):
        k, k1 = jax.random.split(k)
        r1.append(jax.random.randint(k1, (5 * N,), 0, 2147483647))
    r1f = jnp.concatenate(r1)
    start2 = jax.random.randint(jax.random.key(2), (10,), 0, N,
                                dtype=jnp.int32)
    start2p = jnp.concatenate([start2, jnp.zeros((6,), jnp.int32)])
    k = jax.random.key(3)
    r2 = []
    for _ in range(3):
        k, k1 = jax.random.split(k)
        r2.append(jnp.concatenate([jax.random.randint(k1, (10,), 0, 2147483647),
                                   jnp.zeros((6,), jnp.int32)]))
    r2f = jnp.concatenate(r2)

    use_sc_walks = True
    if use_sc_walks:
        xg, wl, coef, = _a2_call(adjb, degb, adjd, degd, start2p, r2f,
                                 r1f, x)
        sl5 = 5 * jnp.arange(40, dtype=jnp.int32)
        wcols = [jnp.full((40,), 5.0, jnp.float32)]
        for p in range(5):
            for t in range(3):
                wcols.append(wl[t * 256 + sl5 + p])
        wt = jnp.stack(wcols, axis=1)
        coefn = coef[:40, None]
    else:
        adjb_m = adjb[:N * MAXDEG].reshape(N, MAXDEG)
        degb_m = degb[:N]
        n2out, coef = _a2w2_call(adjd, degd, start2p, r2f)
        nodes2 = n2out[:40]
        coefn = coef[:40, None]
        v = nodes2
        ucols = [v]
        wcols = [jnp.full((40,), 5.0, jnp.float32)]
        for p in range(5):
            cur = v
            mask = jnp.ones((40,), bool)
            widx = p * N + v
            for t in range(3):
                r = r1[t][widx]
                d = degb_m[cur]
                has = d > 0
                idx = r % jnp.maximum(d, 1)
                nxt = adjb_m[cur, idx]
                mask = mask & has
                cur = jnp.where(mask, nxt, cur)
                ucols.append(cur)
                wcols.append(mask.astype(jnp.float32))
        u = jnp.stack(ucols, axis=1)
        wt = jnp.stack(wcols, axis=1)
        u_pad = jnp.concatenate([u.reshape(-1), jnp.zeros((128,), jnp.int32)])
        (xg,) = _a2t_call(u_pad, x)

    out = _tail(xg, W1, b1[None, :], W3, b3[None, :], wt, coefn)
    return out.reshape(1, 1, DOUT)


# speedup vs baseline: 31.7806x; 31.7806x over previous
import functools

import jax
import jax.numpy as jnp
from jax import lax
from jax.experimental import pallas as pl
from jax.experimental.pallas import tpu as pltpu
from jax.experimental.pallas import tpu_sc as plsc

N = 10000
E = 160000
DIN = 256
DH = 512
DOUT = 256
MAXDEG = 128

NH = 10240
ADJ_SIZE = N * MAXDEG + 128
E1P = 163840
RPT1 = 80
E2P = 327680
RPT2 = 160

_INTERPRET = False


def _zero_ref(ref, nwords):
    z = jnp.zeros((16,), ref.dtype)

    def body(i, _):
        ref[pl.ds(i * 16, 16)] = z
        return 0

    lax.fori_loop(0, nwords // 16, body, 0)


def _build_graph(src2_h, dst2_h, adj_h, deg_h, rpt, vr, sid,
                 src2, dst2, rank2, fidx2, hist, tmp, degacc, sbuf,
                 shared, shared2, sem):
    nv = rpt * 8
    iota = lax.iota(jnp.int32, 16)

    cp_s = pltpu.async_copy(src2_h.at[pl.ds(sid * rpt, rpt)],
                            src2.at[pl.ds(0, rpt)], sem)
    cp_d = pltpu.async_copy(dst2_h.at[pl.ds(sid * rpt, rpt)],
                            dst2.at[pl.ds(0, rpt)], sem)
    cp_s.wait()
    cp_d.wait()

    _zero_ref(hist, NH)

    def p1_one(i):
        row = i // 8
        col = (i % 8) * 16
        s = src2[row, pl.ds(col, 16)]
        ks, lane = plsc.sort_key_val(s * 16 + iota, iota)
        ss = ks >> 4
        sbuf[...] = ss
        prev = plsc.load_gather(sbuf, [jnp.maximum(iota - 1, 0)])
        nxt = plsc.load_gather(sbuf, [jnp.minimum(iota + 1, 15)])
        is_start = (iota == 0) | (ss != prev)
        is_end = (iota == 15) | (ss != nxt)
        sidx = plsc.cummax(jnp.where(is_start, iota, 0))
        runpos = iota - sidx
        old = plsc.load_gather(hist, [ss])
        ranks = old + runpos
        plsc.store_scatter(hist, [ss], ranks + 1, mask=is_end)
        rowv = jnp.zeros((16,), jnp.int32) + row
        plsc.store_scatter(rank2, [rowv, col + lane], ranks)

    def p1(i, _):
        p1_one(2 * i)
        p1_one(2 * i + 1)
        return 0

    with jax.named_scope("p1_ranks"):
        lax.fori_loop(0, nv // 2, p1, 0)

    with jax.named_scope("p2_prefix"):
        pltpu.sync_copy(hist, shared.at[sid])
        plsc.subcore_barrier()
        _zero_ref(degacc, 640)
        for t in range(16):
            pltpu.sync_copy(degacc, shared2.at[t, pl.ds(sid * 640, 640)])
            pltpu.sync_copy(shared.at[t, pl.ds(sid * 640, 640)],
                            tmp.at[pl.ds(0, 640)])
            for j in range(640 // 16):
                degacc[pl.ds(j * 16, 16)] = (degacc[pl.ds(j * 16, 16)]
                                             + tmp[pl.ds(j * 16, 16)])
        for j in range(640 // 16):
            degacc[pl.ds(j * 16, 16)] = jnp.minimum(
                degacc[pl.ds(j * 16, 16)], MAXDEG)
        pltpu.sync_copy(degacc, deg_h.at[pl.ds(sid * 640, 640)])
        plsc.subcore_barrier()
        pltpu.sync_copy(shared2.at[sid], hist)

    def p3(i, _):
        row = i // 8
        col = (i % 8) * 16
        s = src2[row, pl.ds(col, 16)]
        rl = rank2[row, pl.ds(col, 16)]
        rank = rl + plsc.load_gather(hist, [s])
        valid = (s < N) & (rank < MAXDEG)
        f = jnp.where(valid, s * MAXDEG + rank, N * MAXDEG + iota)
        fidx2[row, pl.ds(col, 16)] = f
        return 0

    with jax.named_scope("p3_fidx"):
        lax.fori_loop(0, nv, p3, 0)

    nrow = jnp.clip(vr - sid * rpt, 0, rpt)
    ngrp = nrow // 8

    def pgrp(g, _):
        cps = [pltpu.async_copy(dst2.at[g * 8 + b],
                                adj_h.at[fidx2.at[g * 8 + b]], sem)
               for b in range(8)]
        for cp in cps:
            cp.wait()
        return 0

    with jax.named_scope("p4_scatter"):
        lax.fori_loop(0, ngrp, pgrp, 0)

    def prem(j, _):
        pltpu.async_copy(dst2.at[ngrp * 8 + j],
                         adj_h.at[fidx2.at[ngrp * 8 + j]], sem).wait()
        return 0

    lax.fori_loop(0, nrow - ngrp * 8, prem, 0)


def _a1_body(srcb2_h, dstb2_h, srcd2_h, dstd2_h,
             adjb_h, degb_h, adjd_h, degd_h,
             src2, dst2, rank2, fidx2, hist, tmp, degacc, sbuf,
             shared, shared2, sem):
    cid = lax.axis_index("c")
    sid = lax.axis_index("s")

    @pl.when(cid == 0)
    def _():
        _build_graph(srcb2_h, dstb2_h, adjb_h, degb_h, RPT2, 2 * E // 128, sid,
                     src2, dst2, rank2, fidx2, hist, tmp, degacc, sbuf,
                     shared, shared2, sem)

    @pl.when(cid == 1)
    def _():
        _build_graph(srcd2_h, dstd2_h, adjd_h, degd_h, RPT1, E // 128, sid,
                     src2, dst2, rank2, fidx2, hist, tmp, degacc, sbuf,
                     shared, shared2, sem)


_a1_call = functools.partial(
    pl.kernel,
    out_type=(
        jax.ShapeDtypeStruct((ADJ_SIZE,), jnp.int32),
        jax.ShapeDtypeStruct((NH,), jnp.int32),
        jax.ShapeDtypeStruct((ADJ_SIZE,), jnp.int32),
        jax.ShapeDtypeStruct((NH,), jnp.int32),
    ),
    mesh=plsc.VectorSubcoreMesh(core_axis_name="c", subcore_axis_name="s"),
    scratch_types=[
        pltpu.VMEM((RPT2, 128), jnp.int32),
        pltpu.VMEM((RPT2, 128), jnp.int32),
        pltpu.VMEM((RPT2, 128), jnp.int32),
        pltpu.VMEM((RPT2, 128), jnp.int32),
        pltpu.VMEM((NH,), jnp.int32),
        pltpu.VMEM((NH,), jnp.int32),
        pltpu.VMEM((640,), jnp.int32),
        pltpu.VMEM((16,), jnp.int32),
        pltpu.VMEM_SHARED((16, NH), jnp.int32),
        pltpu.VMEM_SHARED((16, NH), jnp.int32),
        pltpu.SemaphoreType.DMA,
    ],
    compiler_params=pltpu.CompilerParams(needs_layout_passes=False),
)(_a1_body)


def _a2_body(adjb_h, degb_h, adjd_h, degd_h, start2_h, r2_h, r1_h, x_h,
             xg_h, wl_h, coef_h,
             degtab, n2buf, coefbuf, startbuf, rbuf, nxtbuf, ubuf, wbuf,
             ustage, idx48, rows, fbuf, shared_n2, shared_u, sem):
    cid = lax.axis_index("c")
    sid = lax.axis_index("s")
    iota = lax.iota(jnp.int32, 16)

    @pl.when((cid == 0) & (sid == 0))
    def _():
        pltpu.sync_copy(degd_h, degtab)
        pltpu.sync_copy(start2_h, startbuf)
        _zero_ref(coefbuf, 64)
        cur = startbuf[...]
        mask = iota < 10
        one = jnp.full((16,), 1.0, jnp.float32)
        idx0 = jnp.where(mask, iota, 40 + iota - 10)
        plsc.store_scatter(n2buf, [idx0], cur)
        plsc.store_scatter(coefbuf, [idx0], one, mask=mask)
        for t in range(1, 4):
            pltpu.sync_copy(r2_h.at[pl.ds((t - 1) * 16, 16)], rbuf)
            r = rbuf[...]
            d = plsc.load_gather(degtab, [cur])
            has = d > 0
            idx = lax.rem(r, jnp.maximum(d, 1))
            fbuf[...] = cur * MAXDEG + idx
            pltpu.async_copy(adjd_h.at[fbuf], nxtbuf, sem).wait()
            mask = mask & has
            cur = jnp.where(mask, nxtbuf[...], cur)
            idx_t = jnp.where(iota < 10, t * 10 + iota, 40 + iota - 10)
            plsc.store_scatter(n2buf, [idx_t], cur)
            plsc.store_scatter(coefbuf, [idx_t],
                               jnp.where(mask, 1.0, 0.0), mask=iota < 10)
        pltpu.sync_copy(n2buf, shared_n2)
        pltpu.sync_copy(coefbuf.at[pl.ds(0, 48)], coef_h)

    plsc.subcore_barrier()

    @pl.when((cid == 0) & (sid < 13))
    def _():
        pltpu.sync_copy(shared_n2, n2buf)
        pltpu.sync_copy(degb_h, degtab)
        sp = sid * 16 + iota
        valid = sp < 200
        slot = jnp.minimum(sp // 5, 39)
        path = sp % 5
        v = plsc.load_gather(n2buf, [slot])
        ubuf[...] = v
        pltpu.sync_copy(ubuf, shared_u.at[pl.ds(sid * 16, 16)])
        cur = v
        mask = valid
        for t in range(3):
            fbuf[...] = t * 50000 + path * N + v
            pltpu.async_copy(r1_h.at[fbuf], rbuf, sem).wait()
            r = rbuf[...]
            d = plsc.load_gather(degtab, [cur])
            has = d > 0
            idx = lax.rem(r, jnp.maximum(d, 1))
            fbuf[...] = cur * MAXDEG + idx
            pltpu.async_copy(adjb_h.at[fbuf], nxtbuf, sem).wait()
            mask = mask & has
            cur = jnp.where(mask, nxtbuf[...], cur)
            ubuf[...] = cur
            pltpu.sync_copy(ubuf, shared_u.at[pl.ds((1 + t) * 256 + sid * 16, 16)])
            wbuf[...] = jnp.where(mask, 1.0, 0.0)
            pltpu.sync_copy(wbuf, wl_h.at[pl.ds(t * 256 + sid * 16, 16)])

    plsc.subcore_barrier()

    @pl.when(cid == 0)
    def _():
        pltpu.sync_copy(shared_u, ustage)
        for b in range(3):
            jv = sid * 48 + b * 16 + iota
            slot = jnp.minimum(jv >> 4, 39)
            k = jv & 15
            km1 = jnp.maximum(k - 1, 0)
            pos = jnp.where(k == 0, slot * 5,
                            (1 + lax.rem(km1, 3)) * 256 + slot * 5 + km1 // 3)
            vals = plsc.load_gather(ustage, [pos])
            idx48[pl.ds(b * 16, 16)] = vals
        pltpu.async_copy(x_h.at[idx48], rows, sem).wait()
        pltpu.sync_copy(rows, xg_h.at[pl.ds(sid * 48, 48)])


_a2_call = functools.partial(
    pl.kernel,
    out_type=(
        jax.ShapeDtypeStruct((768, DIN), jnp.float32),
        jax.ShapeDtypeStruct((768,), jnp.float32),
        jax.ShapeDtypeStruct((48,), jnp.float32),
    ),
    mesh=plsc.VectorSubcoreMesh(core_axis_name="c", subcore_axis_name="s"),
    scratch_types=[
        pltpu.VMEM((NH,), jnp.int32),
        pltpu.VMEM((64,), jnp.int32),
        pltpu.VMEM((64,), jnp.float32),
        pltpu.VMEM((16,), jnp.int32),
        pltpu.VMEM((16,), jnp.int32),
        pltpu.VMEM((16,), jnp.int32),
        pltpu.VMEM((16,), jnp.int32),
        pltpu.VMEM((16,), jnp.float32),
        pltpu.VMEM((1024,), jnp.int32),
        pltpu.VMEM((48,), jnp.int32),
        pltpu.VMEM((48, DIN), jnp.float32),
        pltpu.VMEM((16,), jnp.int32),
        pltpu.VMEM_SHARED((64,), jnp.int32),
        pltpu.VMEM_SHARED((1024,), jnp.int32),
        pltpu.SemaphoreType.DMA,
    ],
    compiler_params=pltpu.CompilerParams(needs_layout_passes=False),
)(_a2_body)


def _a2w2_body(adjd_h, degd_h, start2_h, r2_h, n2_h, coef_h,
               degtab, n2buf, coefbuf, startbuf, rbuf, nxtbuf, fbuf, sem):
    cid = lax.axis_index("c")
    sid = lax.axis_index("s")
    iota = lax.iota(jnp.int32, 16)

    @pl.when((cid == 0) & (sid == 0))
    def _():
        pltpu.sync_copy(degd_h, degtab)
        pltpu.sync_copy(start2_h, startbuf)
        _zero_ref(coefbuf, 64)
        cur = startbuf[...]
        mask = iota < 10
        one = jnp.full((16,), 1.0, jnp.float32)
        idx0 = jnp.where(mask, iota, 40 + iota - 10)
        plsc.store_scatter(n2buf, [idx0], cur)
        plsc.store_scatter(coefbuf, [idx0], one, mask=mask)
        for t in range(1, 4):
            pltpu.sync_copy(r2_h.at[pl.ds((t - 1) * 16, 16)], rbuf)
            r = rbuf[...]
            d = plsc.load_gather(degtab, [cur])
            has = d > 0
            idx = lax.rem(r, jnp.maximum(d, 1))
            fbuf[...] = cur * MAXDEG + idx
            pltpu.async_copy(adjd_h.at[fbuf], nxtbuf, sem).wait()
            mask = mask & has
            cur = jnp.where(mask, nxtbuf[...], cur)
            idx_t = jnp.where(iota < 10, t * 10 + iota, 40 + iota - 10)
            plsc.store_scatter(n2buf, [idx_t], cur)
            plsc.store_scatter(coefbuf, [idx_t],
                               jnp.where(mask, 1.0, 0.0), mask=iota < 10)
        pltpu.sync_copy(n2buf, n2_h)
        pltpu.sync_copy(coefbuf.at[pl.ds(0, 48)], coef_h)


_a2w2_call = functools.partial(
    pl.kernel,
    out_type=(
        jax.ShapeDtypeStruct((64,), jnp.int32),
        jax.ShapeDtypeStruct((48,), jnp.float32),
    ),
    mesh=plsc.VectorSubcoreMesh(core_axis_name="c", subcore_axis_name="s"),
    scratch_types=[
        pltpu.VMEM((NH,), jnp.int32),
        pltpu.VMEM((64,), jnp.int32),
        pltpu.VMEM((64,), jnp.float32),
        pltpu.VMEM((16,), jnp.int32),
        pltpu.VMEM((16,), jnp.int32),
        pltpu.VMEM((16,), jnp.int32),
        pltpu.VMEM((16,), jnp.int32),
        pltpu.SemaphoreType.DMA,
    ],
    compiler_params=pltpu.CompilerParams(needs_layout_passes=False),
)(_a2w2_body)


def _a2t_body(u_h, x_h, xg_h, idx48, rows, sem):
    cid = lax.axis_index("c")
    sid = lax.axis_index("s")

    @pl.when(cid == 0)
    def _():
        pltpu.sync_copy(u_h.at[pl.ds(sid * 48, 48)], idx48)
        pltpu.async_copy(x_h.at[idx48], rows, sem).wait()
        pltpu.sync_copy(rows, xg_h.at[pl.ds(sid * 48, 48)])


_a2t_call = functools.partial(
    pl.kernel,
    out_type=(jax.ShapeDtypeStruct((768, DIN), jnp.float32),),
    mesh=plsc.VectorSubcoreMesh(core_axis_name="c", subcore_axis_name="s"),
    scratch_types=[
        pltpu.VMEM((48,), jnp.int32),
        pltpu.VMEM((48, DIN), jnp.float32),
        pltpu.SemaphoreType.DMA,
    ],
    compiler_params=pltpu.CompilerParams(needs_layout_passes=False),
)(_a2t_body)


def _tail_body(xg_ref, w1_ref, b1_ref, w3_ref, b3_ref, wt_ref, coefn_ref,
               out_ref):
    xg = xg_ref[...][0:640]
    h = jnp.maximum(
        jnp.dot(xg, w1_ref[...], preferred_element_type=jnp.float32)
        + b1_ref[...], 0.0)
    wt = wt_ref[...]
    cnt = jnp.sum(wt, axis=1, keepdims=True)
    wn = (wt / cnt)[:, :, None]
    a = jnp.sum(h.reshape(40, 16, DH) * wn, axis=1)
    o = jnp.maximum(
        jnp.dot(a, w3_ref[...], preferred_element_type=jnp.float32)
        + b3_ref[...], 0.0)
    coef = coefn_ref[...]
    den = jnp.sum(coef)
    out_ref[...] = jnp.sum(o * (coef / den), axis=0, keepdims=True)


def _tail(xg, w1, b1, w3, b3, wt, coefn):
    return pl.pallas_call(
        _tail_body,
        out_shape=jax.ShapeDtypeStruct((1, DOUT), jnp.float32),
        interpret=_INTERPRET,
    )(xg, w1, b1, w3, b3, wt, coefn)


def kernel(x, edge_index, edge_attr, W1, b1, W2, b2, W3, b3):
    src = edge_index[0].astype(jnp.int32)
    dst = edge_index[1].astype(jnp.int32)
    keep = src != dst
    src_bi = jnp.concatenate([src, jnp.where(keep, dst, jnp.int32(N))])
    dst_bi = jnp.concatenate([dst, jnp.where(keep, src, jnp.int32(0))])

    pad1 = jnp.full((E1P - E,), N, jnp.int32)
    zpad1 = jnp.zeros((E1P - E,), jnp.int32)
    pad2 = jnp.full((E2P - 2 * E,), N, jnp.int32)
    zpad2 = jnp.zeros((E2P - 2 * E,), jnp.int32)
    srcd2 = jnp.concatenate([src, pad1]).reshape(E1P // 128, 128)
    dstd2 = jnp.concatenate([dst, zpad1]).reshape(E1P // 128, 128)
    srcb2 = jnp.concatenate([src_bi, pad2]).reshape(E2P // 128, 128)
    dstb2 = jnp.concatenate([dst_bi, zpad2]).reshape(E2P // 128, 128)

    adjb, degb, adjd, degd = _a1_call(srcb2, dstb2, srcd2, dstd2)

    k = jax.random.key(1)
    r1 = []
    for _ in range(3):
        k, k1 = jax.random.split(k)
        r1.append(jax.random.randint(k1, (5 * N,), 0, 2147483647))
    r1f = jnp.concatenate(r1)
    start2 = jax.random.randint(jax.random.key(2), (10,), 0, N,
                                dtype=jnp.int32)
    start2p = jnp.concatenate([start2, jnp.zeros((6,), jnp.int32)])
    k = jax.random.key(3)
    r2 = []
    for _ in range(3):
        k, k1 = jax.random.split(k)
        r2.append(jnp.concatenate([jax.random.randint(k1, (10,), 0, 2147483647),
                                   jnp.zeros((6,), jnp.int32)]))
    r2f = jnp.concatenate(r2)

    use_sc_walks = True
    if use_sc_walks:
        xg, wl, coef, = _a2_call(adjb, degb, adjd, degd, start2p, r2f,
                                 r1f, x)
        sl5 = 5 * jnp.arange(40, dtype=jnp.int32)
        wcols = [jnp.full((40,), 5.0, jnp.float32)]
        for p in range(5):
            for t in range(3):
                wcols.append(wl[t * 256 + sl5 + p])
        wt = jnp.stack(wcols, axis=1)
        coefn = coef[:40, None]
    else:
        adjb_m = adjb[:N * MAXDEG].reshape(N, MAXDEG)
        degb_m = degb[:N]
        n2out, coef = _a2w2_call(adjd, degd, start2p, r2f)
        nodes2 = n2out[:40]
        coefn = coef[:40, None]
        v = nodes2
        ucols = [v]
        wcols = [jnp.full((40,), 5.0, jnp.float32)]
        for p in range(5):
            cur = v
            mask = jnp.ones((40,), bool)
            widx = p * N + v
            for t in range(3):
                r = r1[t][widx]
                d = degb_m[cur]
                has = d > 0
                idx = r % jnp.maximum(d, 1)
                nxt = adjb_m[cur, idx]
                mask = mask & has
                cur = jnp.where(mask, nxt, cur)
                ucols.append(cur)
                wcols.append(mask.astype(jnp.float32))
        u = jnp.stack(ucols, axis=1)
        wt = jnp.stack(wcols, axis=1)
        u_pad = jnp.concatenate([u.reshape(-1), jnp.zeros((128,), jnp.int32)])
        (xg,) = _a2t_call(u_pad, x)

    out = _tail(xg, W1, b1[None, :], W3, b3[None, :], wt, coefn)
    return out.reshape(1, 1, DOUT)


# speedup vs baseline: 31.7859x; 1.0002x over previous
import functools

import jax
import jax.numpy as jnp
from jax import lax
from jax.experimental import pallas as pl
from jax.experimental.pallas import tpu as pltpu
from jax.experimental.pallas import tpu_sc as plsc

N = 10000
E = 160000
DIN = 256
DH = 512
DOUT = 256
MAXDEG = 128

NH = 10240
ADJ_SIZE = N * MAXDEG + 128
E1P = 163840
RPT1 = 80
E2P = 327680
RPT2 = 160

def _zero_ref(ref, nwords):
    z = jnp.zeros((16,), ref.dtype)

    def body(i, _):
        ref[pl.ds(i * 16, 16)] = z
        return 0

    lax.fori_loop(0, nwords // 16, body, 0)


def _build_graph(src2_h, dst2_h, adj_h, deg_h, rpt, vr, sid,
                 src2, dst2, rank2, fidx2, hist, tmp, degacc, sbuf,
                 shared, shared2, sem):
    nv = rpt * 8
    iota = lax.iota(jnp.int32, 16)

    cp_s = pltpu.async_copy(src2_h.at[pl.ds(sid * rpt, rpt)],
                            src2.at[pl.ds(0, rpt)], sem)
    cp_d = pltpu.async_copy(dst2_h.at[pl.ds(sid * rpt, rpt)],
                            dst2.at[pl.ds(0, rpt)], sem)
    cp_s.wait()
    cp_d.wait()

    _zero_ref(hist, NH)

    def p1_one(i):
        row = i // 8
        col = (i % 8) * 16
        s = src2[row, pl.ds(col, 16)]
        ks, lane = plsc.sort_key_val(s * 16 + iota, iota)
        ss = ks >> 4
        sbuf[...] = ss
        prev = plsc.load_gather(sbuf, [jnp.maximum(iota - 1, 0)])
        nxt = plsc.load_gather(sbuf, [jnp.minimum(iota + 1, 15)])
        is_start = (iota == 0) | (ss != prev)
        is_end = (iota == 15) | (ss != nxt)
        sidx = plsc.cummax(jnp.where(is_start, iota, 0))
        runpos = iota - sidx
        old = plsc.load_gather(hist, [ss])
        ranks = old + runpos
        plsc.store_scatter(hist, [ss], ranks + 1, mask=is_end)
        rowv = jnp.zeros((16,), jnp.int32) + row
        plsc.store_scatter(rank2, [rowv, col + lane], ranks)

    def p1(i, _):
        p1_one(2 * i)
        p1_one(2 * i + 1)
        return 0

    with jax.named_scope("p1_ranks"):
        lax.fori_loop(0, nv // 2, p1, 0)

    with jax.named_scope("p2_prefix"):
        pltpu.sync_copy(hist, shared.at[sid])
        plsc.subcore_barrier()
        _zero_ref(degacc, 640)
        for t in range(16):
            pltpu.sync_copy(degacc, shared2.at[t, pl.ds(sid * 640, 640)])
            pltpu.sync_copy(shared.at[t, pl.ds(sid * 640, 640)],
                            tmp.at[pl.ds(0, 640)])
            for j in range(640 // 16):
                degacc[pl.ds(j * 16, 16)] = (degacc[pl.ds(j * 16, 16)]
                                             + tmp[pl.ds(j * 16, 16)])
        for j in range(640 // 16):
            degacc[pl.ds(j * 16, 16)] = jnp.minimum(
                degacc[pl.ds(j * 16, 16)], MAXDEG)
        pltpu.sync_copy(degacc, deg_h.at[pl.ds(sid * 640, 640)])
        plsc.subcore_barrier()
        pltpu.sync_copy(shared2.at[sid], hist)

    def p3(i, _):
        row = i // 8
        col = (i % 8) * 16
        s = src2[row, pl.ds(col, 16)]
        rl = rank2[row, pl.ds(col, 16)]
        rank = rl + plsc.load_gather(hist, [s])
        valid = (s < N) & (rank < MAXDEG)
        f = jnp.where(valid, s * MAXDEG + rank, N * MAXDEG + iota)
        fidx2[row, pl.ds(col, 16)] = f
        return 0

    with jax.named_scope("p3_fidx"):
        lax.fori_loop(0, nv, p3, 0)

    nrow = jnp.clip(vr - sid * rpt, 0, rpt)
    ngrp = nrow // 8

    def pgrp(g, _):
        cps = [pltpu.async_copy(dst2.at[g * 8 + b],
                                adj_h.at[fidx2.at[g * 8 + b]], sem)
               for b in range(8)]
        for cp in cps:
            cp.wait()
        return 0

    with jax.named_scope("p4_scatter"):
        lax.fori_loop(0, ngrp, pgrp, 0)

    def prem(j, _):
        pltpu.async_copy(dst2.at[ngrp * 8 + j],
                         adj_h.at[fidx2.at[ngrp * 8 + j]], sem).wait()
        return 0

    lax.fori_loop(0, nrow - ngrp * 8, prem, 0)


def _a1_body(srcb2_h, dstb2_h, srcd2_h, dstd2_h,
             adjb_h, degb_h, adjd_h, degd_h,
             src2, dst2, rank2, fidx2, hist, tmp, degacc, sbuf,
             shared, shared2, sem):
    cid = lax.axis_index("c")
    sid = lax.axis_index("s")

    @pl.when(cid == 0)
    def _():
        _build_graph(srcb2_h, dstb2_h, adjb_h, degb_h, RPT2, 2 * E // 128, sid,
                     src2, dst2, rank2, fidx2, hist, tmp, degacc, sbuf,
                     shared, shared2, sem)

    @pl.when(cid == 1)
    def _():
        _build_graph(srcd2_h, dstd2_h, adjd_h, degd_h, RPT1, E // 128, sid,
                     src2, dst2, rank2, fidx2, hist, tmp, degacc, sbuf,
                     shared, shared2, sem)


_a1_call = functools.partial(
    pl.kernel,
    out_type=(
        jax.ShapeDtypeStruct((ADJ_SIZE,), jnp.int32),
        jax.ShapeDtypeStruct((NH,), jnp.int32),
        jax.ShapeDtypeStruct((ADJ_SIZE,), jnp.int32),
        jax.ShapeDtypeStruct((NH,), jnp.int32),
    ),
    mesh=plsc.VectorSubcoreMesh(core_axis_name="c", subcore_axis_name="s"),
    scratch_types=[
        pltpu.VMEM((RPT2, 128), jnp.int32),
        pltpu.VMEM((RPT2, 128), jnp.int32),
        pltpu.VMEM((RPT2, 128), jnp.int32),
        pltpu.VMEM((RPT2, 128), jnp.int32),
        pltpu.VMEM((NH,), jnp.int32),
        pltpu.VMEM((NH,), jnp.int32),
        pltpu.VMEM((640,), jnp.int32),
        pltpu.VMEM((16,), jnp.int32),
        pltpu.VMEM_SHARED((16, NH), jnp.int32),
        pltpu.VMEM_SHARED((16, NH), jnp.int32),
        pltpu.SemaphoreType.DMA,
    ],
    compiler_params=pltpu.CompilerParams(needs_layout_passes=False),
)(_a1_body)


def _a2_body(adjb_h, degb_h, adjd_h, degd_h, start2_h, r2_h, r1_h, x_h,
             xg_h, wl_h, coef_h,
             degtab, n2buf, coefbuf, startbuf, rbuf, nxtbuf, ubuf, wbuf,
             ustage, idx48, rows, fbuf, shared_n2, shared_u, sem):
    cid = lax.axis_index("c")
    sid = lax.axis_index("s")
    iota = lax.iota(jnp.int32, 16)

    @pl.when((cid == 0) & (sid == 0))
    def _():
        pltpu.sync_copy(degd_h, degtab)
        pltpu.sync_copy(start2_h, startbuf)
        _zero_ref(coefbuf, 64)
        cur = startbuf[...]
        mask = iota < 10
        one = jnp.full((16,), 1.0, jnp.float32)
        idx0 = jnp.where(mask, iota, 40 + iota - 10)
        plsc.store_scatter(n2buf, [idx0], cur)
        plsc.store_scatter(coefbuf, [idx0], one, mask=mask)
        for t in range(1, 4):
            pltpu.sync_copy(r2_h.at[pl.ds((t - 1) * 16, 16)], rbuf)
            r = rbuf[...]
            d = plsc.load_gather(degtab, [cur])
            has = d > 0
            idx = lax.rem(r, jnp.maximum(d, 1))
            fbuf[...] = cur * MAXDEG + idx
            pltpu.async_copy(adjd_h.at[fbuf], nxtbuf, sem).wait()
            mask = mask & has
            cur = jnp.where(mask, nxtbuf[...], cur)
            idx_t = jnp.where(iota < 10, t * 10 + iota, 40 + iota - 10)
            plsc.store_scatter(n2buf, [idx_t], cur)
            plsc.store_scatter(coefbuf, [idx_t],
                               jnp.where(mask, 1.0, 0.0), mask=iota < 10)
        pltpu.sync_copy(n2buf, shared_n2)
        pltpu.sync_copy(coefbuf.at[pl.ds(0, 48)], coef_h)

    plsc.subcore_barrier()

    @pl.when((cid == 0) & (sid < 13))
    def _():
        pltpu.sync_copy(shared_n2, n2buf)
        pltpu.sync_copy(degb_h, degtab)
        sp = sid * 16 + iota
        valid = sp < 200
        slot = jnp.minimum(sp // 5, 39)
        path = sp % 5
        v = plsc.load_gather(n2buf, [slot])
        ubuf[...] = v
        pltpu.sync_copy(ubuf, shared_u.at[pl.ds(sid * 16, 16)])
        cur = v
        mask = valid
        for t in range(3):
            fbuf[...] = t * 50000 + path * N + v
            pltpu.async_copy(r1_h.at[fbuf], rbuf, sem).wait()
            r = rbuf[...]
            d = plsc.load_gather(degtab, [cur])
            has = d > 0
            idx = lax.rem(r, jnp.maximum(d, 1))
            fbuf[...] = cur * MAXDEG + idx
            pltpu.async_copy(adjb_h.at[fbuf], nxtbuf, sem).wait()
            mask = mask & has
            cur = jnp.where(mask, nxtbuf[...], cur)
            ubuf[...] = cur
            pltpu.sync_copy(ubuf, shared_u.at[pl.ds((1 + t) * 256 + sid * 16, 16)])
            wbuf[...] = jnp.where(mask, 1.0, 0.0)
            pltpu.sync_copy(wbuf, wl_h.at[pl.ds(t * 256 + sid * 16, 16)])

    plsc.subcore_barrier()

    @pl.when(cid == 0)
    def _():
        pltpu.sync_copy(shared_u, ustage)
        for b in range(3):
            jv = sid * 48 + b * 16 + iota
            slot = jnp.minimum(jv >> 4, 39)
            k = jv & 15
            km1 = jnp.maximum(k - 1, 0)
            pos = jnp.where(k == 0, slot * 5,
                            (1 + lax.rem(km1, 3)) * 256 + slot * 5 + km1 // 3)
            vals = plsc.load_gather(ustage, [pos])
            idx48[pl.ds(b * 16, 16)] = vals
        pltpu.async_copy(x_h.at[idx48], rows, sem).wait()
        pltpu.sync_copy(rows, xg_h.at[pl.ds(sid * 48, 48)])


_a2_call = functools.partial(
    pl.kernel,
    out_type=(
        jax.ShapeDtypeStruct((768, DIN), jnp.float32),
        jax.ShapeDtypeStruct((768,), jnp.float32),
        jax.ShapeDtypeStruct((48,), jnp.float32),
    ),
    mesh=plsc.VectorSubcoreMesh(core_axis_name="c", subcore_axis_name="s"),
    scratch_types=[
        pltpu.VMEM((NH,), jnp.int32),
        pltpu.VMEM((64,), jnp.int32),
        pltpu.VMEM((64,), jnp.float32),
        pltpu.VMEM((16,), jnp.int32),
        pltpu.VMEM((16,), jnp.int32),
        pltpu.VMEM((16,), jnp.int32),
        pltpu.VMEM((16,), jnp.int32),
        pltpu.VMEM((16,), jnp.float32),
        pltpu.VMEM((1024,), jnp.int32),
        pltpu.VMEM((48,), jnp.int32),
        pltpu.VMEM((48, DIN), jnp.float32),
        pltpu.VMEM((16,), jnp.int32),
        pltpu.VMEM_SHARED((64,), jnp.int32),
        pltpu.VMEM_SHARED((1024,), jnp.int32),
        pltpu.SemaphoreType.DMA,
    ],
    compiler_params=pltpu.CompilerParams(needs_layout_passes=False),
)(_a2_body)


def _tail_body(xg_ref, w1_ref, b1_ref, w3_ref, b3_ref, wt_ref, coefn_ref,
               out_ref):
    xg = xg_ref[...][0:640]
    h = jnp.maximum(
        jnp.dot(xg, w1_ref[...], preferred_element_type=jnp.float32)
        + b1_ref[...], 0.0)
    wt = wt_ref[...]
    cnt = jnp.sum(wt, axis=1, keepdims=True)
    wn = (wt / cnt)[:, :, None]
    a = jnp.sum(h.reshape(40, 16, DH) * wn, axis=1)
    o = jnp.maximum(
        jnp.dot(a, w3_ref[...], preferred_element_type=jnp.float32)
        + b3_ref[...], 0.0)
    coef = coefn_ref[...]
    den = jnp.sum(coef)
    out_ref[...] = jnp.sum(o * (coef / den), axis=0, keepdims=True)


def _tail(xg, w1, b1, w3, b3, wt, coefn):
    return pl.pallas_call(
        _tail_body,
        out_shape=jax.ShapeDtypeStruct((1, DOUT), jnp.float32),
    )(xg, w1, b1, w3, b3, wt, coefn)


def kernel(x, edge_index, edge_attr, W1, b1, W2, b2, W3, b3):
    src = edge_index[0].astype(jnp.int32)
    dst = edge_index[1].astype(jnp.int32)
    keep = src != dst
    src_bi = jnp.concatenate([src, jnp.where(keep, dst, jnp.int32(N))])
    dst_bi = jnp.concatenate([dst, jnp.where(keep, src, jnp.int32(0))])

    pad1 = jnp.full((E1P - E,), N, jnp.int32)
    zpad1 = jnp.zeros((E1P - E,), jnp.int32)
    pad2 = jnp.full((E2P - 2 * E,), N, jnp.int32)
    zpad2 = jnp.zeros((E2P - 2 * E,), jnp.int32)
    srcd2 = jnp.concatenate([src, pad1]).reshape(E1P // 128, 128)
    dstd2 = jnp.concatenate([dst, zpad1]).reshape(E1P // 128, 128)
    srcb2 = jnp.concatenate([src_bi, pad2]).reshape(E2P // 128, 128)
    dstb2 = jnp.concatenate([dst_bi, zpad2]).reshape(E2P // 128, 128)

    adjb, degb, adjd, degd = _a1_call(srcb2, dstb2, srcd2, dstd2)

    k = jax.random.key(1)
    r1 = []
    for _ in range(3):
        k, k1 = jax.random.split(k)
        r1.append(jax.random.randint(k1, (5 * N,), 0, 2147483647))
    r1f = jnp.concatenate(r1)
    start2 = jax.random.randint(jax.random.key(2), (10,), 0, N,
                                dtype=jnp.int32)
    start2p = jnp.concatenate([start2, jnp.zeros((6,), jnp.int32)])
    k = jax.random.key(3)
    r2 = []
    for _ in range(3):
        k, k1 = jax.random.split(k)
        r2.append(jnp.concatenate([jax.random.randint(k1, (10,), 0, 2147483647),
                                   jnp.zeros((6,), jnp.int32)]))
    r2f = jnp.concatenate(r2)

    xg, wl, coef, = _a2_call(adjb, degb, adjd, degd, start2p, r2f,
                             r1f, x)
    sl5 = 5 * jnp.arange(40, dtype=jnp.int32)
    wcols = [jnp.full((40,), 5.0, jnp.float32)]
    for p in range(5):
        for t in range(3):
            wcols.append(wl[t * 256 + sl5 + p])
    wt = jnp.stack(wcols, axis=1)
    coefn = coef[:40, None]

    out = _tail(xg, W1, b1[None, :], W3, b3[None, :], wt, coefn)
    return out.reshape(1, 1, DOUT)
